# baseline (speedup 1.0000x reference)
import functools

import jax
import jax.numpy as jnp
from jax import lax
from jax.experimental import pallas as pl
from jax.experimental.pallas import tpu as pltpu
from jax.experimental.pallas import tpu_sc as plsc

_INFO = plsc.get_sparse_core_info()
_NC = _INFO.num_cores
_NS = _INFO.num_subcores
_NW = _NC * _NS
_CW = 128


def _embed_lookup(NJ, NI, D):
    mesh = plsc.VectorSubcoreMesh(core_axis_name="c", subcore_axis_name="s")
    nib = NI // _CW
    ib_per_w = nib // _NW
    n_chunks = NJ * ib_per_w
    ncb = D // 8

    @functools.partial(
        pl.kernel,
        mesh=mesh,
        out_type=jax.ShapeDtypeStruct((NJ, ncb, nib, 8, _CW), jnp.float32),
        scratch_types=[
            pltpu.VMEM((NJ, ib_per_w * _CW), jnp.int32),
            pltpu.VMEM((2, _CW, D), jnp.float32),
            pltpu.VMEM((2, ncb, 8, _CW), jnp.float32),
            pltpu.SemaphoreType.DMA,
            pltpu.SemaphoreType.DMA,
        ],
        compiler_params=pltpu.CompilerParams(
            use_tc_tiling_on_sc=False, needs_layout_passes=False),
    )
    def k(idxT_hbm, table_hbm, out_hbm, idx_v, rows_v, tile_v, gsem, ssem):
        wid = lax.axis_index("s") * _NC + lax.axis_index("c")
        ib0 = wid * ib_per_w
        pltpu.sync_copy(idxT_hbm.at[:, pl.ds(ib0 * _CW, ib_per_w * _CW)], idx_v)

        rvecs = [jnp.arange(16, dtype=jnp.int32) + (lg * 16) for lg in range(8)]

        def coords(g):
            return g // ib_per_w, g % ib_per_w

        def issue_gather(g, b):
            jj, ibl = coords(g)
            pltpu.async_copy(
                table_hbm.at[idx_v.at[jj, pl.ds(ibl * _CW, _CW)]],
                rows_v.at[b], gsem)

        def wait_gather(g, b):
            jj, ibl = coords(g)
            pltpu.make_async_copy(
                table_hbm.at[idx_v.at[jj, pl.ds(ibl * _CW, _CW)]],
                rows_v.at[b], gsem).wait()

        def issue_store(g, b):
            jj, ibl = coords(g)
            pltpu.async_copy(tile_v.at[b], out_hbm.at[jj, :, ib0 + ibl], ssem)

        def wait_store(g, b):
            jj, ibl = coords(g)
            pltpu.make_async_copy(
                tile_v.at[b], out_hbm.at[jj, :, ib0 + ibl], ssem).wait()

        def transpose(b):
            for c in range(D):
                cvec = jnp.full((16,), c, dtype=jnp.int32)
                for lg in range(8):
                    vals = plsc.load_gather(rows_v.at[b], [rvecs[lg], cvec])
                    tile_v[b, c // 8, c % 8, pl.ds(lg * 16, 16)] = vals

        issue_gather(0, 0)

        def body(gg, carry):
            for b in (0, 1):
                g = 2 * gg + b

                @pl.when(g + 1 < n_chunks)
                def _():
                    issue_gather(g + 1, 1 - b)

                wait_gather(g, b)

                @pl.when(g >= 2)
                def _():
                    wait_store(g - 2, b)

                transpose(b)
                issue_store(g, b)
            return carry

        lax.fori_loop(0, n_chunks // 2, body, 0)
        wait_store(n_chunks - 2, 0)
        wait_store(n_chunks - 1, 1)

    return k


def kernel(token_ids, weight):
    NI, NJ = token_ids.shape
    V, D = weight.shape
    idxT = jnp.transpose(token_ids).astype(jnp.int32)
    out5 = _embed_lookup(NJ, NI, D)(idxT, weight)
    return out5.transpose((2, 4, 0, 1, 3)).reshape(NI, NJ, D)


# speedup vs baseline: 1.2680x; 1.2680x over previous
import functools

import jax
import jax.numpy as jnp
from jax import lax
from jax.experimental import pallas as pl
from jax.experimental.pallas import tpu as pltpu
from jax.experimental.pallas import tpu_sc as plsc

_INFO = plsc.get_sparse_core_info()
_NC = _INFO.num_cores
_NS = _INFO.num_subcores
_NW = _NC * _NS
_CW = 128


def _embed_lookup(NJ, NI, D):
    mesh = plsc.VectorSubcoreMesh(core_axis_name="c", subcore_axis_name="s")
    nib = NI // _CW
    ib_per_w = nib // _NW
    n_chunks = NJ * ib_per_w
    ncb = D // 8

    @functools.partial(
        pl.kernel,
        mesh=mesh,
        out_type=jax.ShapeDtypeStruct((NJ, ncb, nib, 8, _CW), jnp.float32),
        scratch_types=[
            pltpu.VMEM((NJ, ib_per_w * _CW), jnp.int32),
            pltpu.VMEM((2, _CW, D), jnp.float32),
            pltpu.VMEM((2, ncb, 8, _CW), jnp.float32),
            pltpu.SemaphoreType.DMA,
            pltpu.SemaphoreType.DMA,
        ],
        compiler_params=pltpu.CompilerParams(
            use_tc_tiling_on_sc=False, needs_layout_passes=False),
    )
    def k(idxT_hbm, table_hbm, out_hbm, idx_v, rows_v, tile_v, gsem, ssem):
        wid = lax.axis_index("s") * _NC + lax.axis_index("c")
        ib0 = wid * ib_per_w
        pltpu.sync_copy(idxT_hbm.at[:, pl.ds(ib0 * _CW, ib_per_w * _CW)], idx_v)

        rvecs = [jnp.arange(16, dtype=jnp.int32) + (lg * 16) for lg in range(8)]

        def coords(g):
            return g // ib_per_w, g % ib_per_w

        def issue_gather(g, b):
            jj, ibl = coords(g)
            pltpu.async_copy(
                table_hbm.at[idx_v.at[jj, pl.ds(ibl * _CW, _CW)]],
                rows_v.at[b], gsem)

        def wait_gather(g, b):
            jj, ibl = coords(g)
            pltpu.make_async_copy(
                table_hbm.at[idx_v.at[jj, pl.ds(ibl * _CW, _CW)]],
                rows_v.at[b], gsem).wait()

        def issue_store(g, b):
            jj, ibl = coords(g)
            pltpu.async_copy(tile_v.at[b], out_hbm.at[jj, :, ib0 + ibl], ssem)

        def wait_store(g, b):
            jj, ibl = coords(g)
            pltpu.make_async_copy(
                tile_v.at[b], out_hbm.at[jj, :, ib0 + ibl], ssem).wait()

        def transpose(b):
            def cbody(c, carry):
                cvec = jnp.full((16,), 1, dtype=jnp.int32) * c
                vals = [plsc.load_gather(rows_v.at[b], [rvecs[lg], cvec])
                        for lg in range(8)]
                for lg in range(8):
                    tile_v[b, c // 8, c % 8, pl.ds(lg * 16, 16)] = vals[lg]
                return carry

            lax.fori_loop(0, D, cbody, 0)

        issue_gather(0, 0)

        def body(gg, carry):
            for b in (0, 1):
                g = 2 * gg + b

                @pl.when(g + 1 < n_chunks)
                def _():
                    issue_gather(g + 1, 1 - b)

                wait_gather(g, b)

                @pl.when(g >= 2)
                def _():
                    wait_store(g - 2, b)

                transpose(b)
                issue_store(g, b)
            return carry

        lax.fori_loop(0, n_chunks // 2, body, 0)
        wait_store(n_chunks - 2, 0)
        wait_store(n_chunks - 1, 1)

    return k


def kernel(token_ids, weight):
    NI, NJ = token_ids.shape
    V, D = weight.shape
    idxT = jnp.transpose(token_ids).astype(jnp.int32)
    out5 = _embed_lookup(NJ, NI, D)(idxT, weight)
    return out5.transpose((2, 4, 0, 1, 3)).reshape(NI, NJ, D)


# speedup vs baseline: 1.2694x; 1.0011x over previous
import functools

import jax
import jax.numpy as jnp
from jax import lax
from jax.experimental import pallas as pl
from jax.experimental.pallas import tpu as pltpu
from jax.experimental.pallas import tpu_sc as plsc

_INFO = plsc.get_sparse_core_info()
_NC = _INFO.num_cores
_NS = _INFO.num_subcores
_NW = _NC * _NS
_CW = 128


def _embed_lookup(NJ, NI, D):
    mesh = plsc.VectorSubcoreMesh(core_axis_name="c", subcore_axis_name="s")
    nib = NI // _CW
    ib_per_w = nib // _NW
    n_chunks = NJ * ib_per_w
    ncb = D // 8

    @functools.partial(
        pl.kernel,
        mesh=mesh,
        out_type=jax.ShapeDtypeStruct((NJ, ncb, nib, 8, _CW), jnp.float32),
        scratch_types=[
            pltpu.VMEM((NJ, ib_per_w * _CW), jnp.int32),
            pltpu.VMEM((4, _CW, D), jnp.float32),
            pltpu.VMEM((4, ncb, 8, _CW), jnp.float32),
            pltpu.SemaphoreType.DMA,
            pltpu.SemaphoreType.DMA,
        ],
        compiler_params=pltpu.CompilerParams(
            use_tc_tiling_on_sc=False, needs_layout_passes=False),
    )
    def k(idxT_hbm, table_hbm, out_hbm, idx_v, rows_v, tile_v, gsem, ssem):
        wid = lax.axis_index("s") * _NC + lax.axis_index("c")
        ib0 = wid * ib_per_w
        pltpu.sync_copy(idxT_hbm.at[:, pl.ds(ib0 * _CW, ib_per_w * _CW)], idx_v)

        rvecs = [jnp.arange(16, dtype=jnp.int32) + (lg * 16) for lg in range(8)]

        def coords(g):
            return g // ib_per_w, g % ib_per_w

        def issue_gather(g, b):
            jj, ibl = coords(g)
            pltpu.async_copy(
                table_hbm.at[idx_v.at[jj, pl.ds(ibl * _CW, _CW)]],
                rows_v.at[b], gsem)

        def wait_gather(g, b):
            jj, ibl = coords(g)
            pltpu.make_async_copy(
                table_hbm.at[idx_v.at[jj, pl.ds(ibl * _CW, _CW)]],
                rows_v.at[b], gsem).wait()

        def issue_store(g, b):
            jj, ibl = coords(g)
            pltpu.async_copy(tile_v.at[b], out_hbm.at[jj, :, ib0 + ibl], ssem)

        def wait_store(g, b):
            jj, ibl = coords(g)
            pltpu.make_async_copy(
                tile_v.at[b], out_hbm.at[jj, :, ib0 + ibl], ssem).wait()

        def transpose(b):
            def cbody(c, carry):
                cvec = jnp.full((16,), 1, dtype=jnp.int32) * c
                vals = [plsc.load_gather(rows_v.at[b], [rvecs[lg], cvec])
                        for lg in range(8)]
                for lg in range(8):
                    tile_v[b, c // 8, c % 8, pl.ds(lg * 16, 16)] = vals[lg]
                return carry

            lax.fori_loop(0, D, cbody, 0)

        for g0 in range(3):
            issue_gather(g0, g0)

        def body(gg, carry):
            for b in range(4):
                g = 4 * gg + b

                @pl.when(g + 3 < n_chunks)
                def _():
                    issue_gather(g + 3, (b + 3) % 4)

                wait_gather(g, b)

                @pl.when(g >= 4)
                def _():
                    wait_store(g - 4, b)

                transpose(b)
                issue_store(g, b)
            return carry

        lax.fori_loop(0, n_chunks // 4, body, 0)
        for g0 in range(4):
            wait_store(n_chunks - 4 + g0, g0)

    return k


def kernel(token_ids, weight):
    NI, NJ = token_ids.shape
    V, D = weight.shape
    idxT = jnp.transpose(token_ids).astype(jnp.int32)
    out5 = _embed_lookup(NJ, NI, D)(idxT, weight)
    return out5.transpose((2, 4, 0, 1, 3)).reshape(NI, NJ, D)


# speedup vs baseline: 2.6671x; 2.1011x over previous
import functools

import jax
import jax.numpy as jnp
from jax import lax
from jax.experimental import pallas as pl
from jax.experimental.pallas import tpu as pltpu
from jax.experimental.pallas import tpu_sc as plsc

_INFO = plsc.get_sparse_core_info()
_NC = _INFO.num_cores
_NS = _INFO.num_subcores
_NW = _NC * _NS
_CW = 128


def _embed_lookup(NJ, NI, D):
    mesh = plsc.VectorSubcoreMesh(core_axis_name="c", subcore_axis_name="s")
    nib = NI // _CW
    ib_per_w = nib // _NW
    n_chunks = NJ * ib_per_w
    ncb = D // 8

    @functools.partial(
        pl.kernel,
        mesh=mesh,
        out_type=jax.ShapeDtypeStruct((NJ, ncb, nib, 8, _CW), jnp.float32),
        scratch_types=[
            pltpu.VMEM((NJ, ib_per_w * _CW), jnp.int32),
            pltpu.VMEM((4, _CW, D), jnp.float32),
            pltpu.VMEM((4, D, _CW), jnp.float32),
            pltpu.SemaphoreType.DMA,
            pltpu.SemaphoreType.DMA,
        ],
        compiler_params=pltpu.CompilerParams(
            use_tc_tiling_on_sc=False, needs_layout_passes=False),
    )
    def k(idxT_hbm, table_hbm, out_hbm, idx_v, rows_v, tile_v, gsem, ssem):
        wid = lax.axis_index("s") * _NC + lax.axis_index("c")
        ib0 = wid * ib_per_w
        pltpu.sync_copy(idxT_hbm.at[:, pl.ds(ib0 * _CW, ib_per_w * _CW)], idx_v)

        iota = jnp.arange(16, dtype=jnp.int32)
        lvecs = [iota + (lg * 16) for lg in range(8)]
        perms = [(iota + i) & 15 for i in range(16)]

        def coords(g):
            return g // ib_per_w, g % ib_per_w

        def issue_gather(g, b):
            jj, ibl = coords(g)
            pltpu.async_copy(
                table_hbm.at[idx_v.at[jj, pl.ds(ibl * _CW, _CW)]],
                rows_v.at[b], gsem)

        def wait_gather(g, b):
            jj, ibl = coords(g)
            pltpu.make_async_copy(
                table_hbm.at[idx_v.at[jj, pl.ds(ibl * _CW, _CW)]],
                rows_v.at[b], gsem).wait()

        def issue_store(g, b):
            jj, ibl = coords(g)
            for cb in range(ncb):
                pltpu.async_copy(
                    tile_v.at[b, pl.ds(cb * 8, 8)],
                    out_hbm.at[jj, cb, ib0 + ibl], ssem)

        def wait_store(g, b):
            jj, ibl = coords(g)
            for cb in range(ncb):
                pltpu.make_async_copy(
                    tile_v.at[b, pl.ds(cb * 8, 8)],
                    out_hbm.at[jj, cb, ib0 + ibl], ssem).wait()

        def transpose(b):
            def cqbody(cq, carry):
                cvecs = [perms[i] + cq * 16 for i in range(16)]
                for lg in range(8):
                    vals = [plsc.load_gather(rows_v.at[b], [lvecs[lg], cvecs[i]])
                            for i in range(16)]
                    for i in range(16):
                        plsc.store_scatter(
                            tile_v.at[b], [cvecs[i], lvecs[lg]], vals[i])
                return carry

            lax.fori_loop(0, D // 16, cqbody, 0)

        for g0 in range(3):
            issue_gather(g0, g0)

        def body(gg, carry):
            for b in range(4):
                g = 4 * gg + b

                @pl.when(g + 3 < n_chunks)
                def _():
                    issue_gather(g + 3, (b + 3) % 4)

                wait_gather(g, b)

                @pl.when(g >= 4)
                def _():
                    wait_store(g - 4, b)

                transpose(b)
                issue_store(g, b)
            return carry

        lax.fori_loop(0, n_chunks // 4, body, 0)
        for g0 in range(4):
            wait_store(n_chunks - 4 + g0, g0)

    return k


def kernel(token_ids, weight):
    NI, NJ = token_ids.shape
    V, D = weight.shape
    idxT = jnp.transpose(token_ids).astype(jnp.int32)
    out5 = _embed_lookup(NJ, NI, D)(idxT, weight)
    return out5.transpose((2, 4, 0, 1, 3)).reshape(NI, NJ, D)
